# baseline (speedup 1.0000x reference)
import functools

import jax
import jax.numpy as jnp
from jax.experimental import pallas as pl
from jax.experimental.pallas import tpu as pltpu

HIDDEN = 128
BLOCK_ROWS = 1024


def _broadcast_block(table_ref, out_ref):
    out_ref[...] = jnp.broadcast_to(table_ref[...], out_ref.shape)


def kernel(x, table):
    batch = x.shape[0]
    return pl.pallas_call(
        _broadcast_block,
        grid=(batch // BLOCK_ROWS,),
        in_specs=[pl.BlockSpec((1, HIDDEN), lambda i: (0, 0))],
        out_specs=pl.BlockSpec((BLOCK_ROWS, HIDDEN), lambda i: (i, 0)),
        out_shape=jax.ShapeDtypeStruct((batch, HIDDEN), jnp.float32),
        compiler_params=pltpu.CompilerParams(
            dimension_semantics=("parallel",),
        ),
    )(table.astype(jnp.float32))


# speedup vs baseline: 1.9710x; 1.9710x over previous
import jax
import jax.numpy as jnp
from jax.experimental import pallas as pl
from jax.experimental.pallas import tpu as pltpu

HIDDEN = 128
BLOCK_ROWS = 256


def _broadcast_body(table_ref, out_ref, scratch, sem):
    scratch[...] = jnp.broadcast_to(table_ref[...], scratch.shape)
    batch = out_ref.shape[0]
    copies = []
    for t in range(batch // BLOCK_ROWS):
        c = pltpu.make_async_copy(
            scratch, out_ref.at[pl.ds(t * BLOCK_ROWS, BLOCK_ROWS)], sem
        )
        c.start()
        copies.append(c)
    for c in copies:
        c.wait()


def kernel(x, table):
    batch = x.shape[0]
    return pl.pallas_call(
        _broadcast_body,
        in_specs=[pl.BlockSpec(memory_space=pltpu.VMEM)],
        out_specs=pl.BlockSpec(memory_space=pl.ANY),
        out_shape=jax.ShapeDtypeStruct((batch, HIDDEN), jnp.float32),
        scratch_shapes=[
            pltpu.VMEM((BLOCK_ROWS, HIDDEN), jnp.float32),
            pltpu.SemaphoreType.DMA,
        ],
    )(table.astype(jnp.float32))
